# baseline (speedup 1.0000x reference)
import functools

import jax
import jax.numpy as jnp
from jax import lax
from jax.experimental import pallas as pl
from jax.experimental.pallas import tpu as pltpu
from jax.experimental.pallas import tpu_sc as plsc

HEADS = 8
NEG_SLOPE = 0.2
_GDN = lax.GatherDimensionNumbers(
    offset_dims=(), collapsed_slice_dims=(0,), start_index_map=(0,))


def _splat(vec, i):
    idx = jnp.full((16, 1), i, dtype=jnp.int32)
    return lax.gather(vec, idx, _GDN, (1,),
                      mode=lax.GatherScatterMode.PROMISE_IN_BOUNDS)


def _lrelu_exp(z):
    return jnp.exp(jnp.maximum(z, NEG_SLOPE * z))


def _gat_sc(h_p, asrcT_flat, adstT_flat, nn_flat, bias):
    NPAD, C_out = h_p.shape
    N = 10000
    ROW = 144
    SPW = NPAD // 16
    NBLK = SPW // 16
    mesh = plsc.VectorSubcoreMesh(core_axis_name="c", subcore_axis_name="s")

    @functools.partial(
        pl.kernel,
        mesh=mesh,
        compiler_params=pltpu.CompilerParams(
            needs_layout_passes=False, use_tc_tiling_on_sc=False),
        out_type=jax.ShapeDtypeStruct((N, C_out), jnp.float32),
        scratch_types=[
            pltpu.VMEM_SHARED((N, ROW), jnp.float32),
            pltpu.VMEM((256,), jnp.int32),
            pltpu.VMEM((2 * N,), jnp.float32),
            pltpu.VMEM((2, SPW), jnp.float32),
            pltpu.VMEM((16, 128), jnp.float32),
            pltpu.VMEM((16, ROW), jnp.float32),
            pltpu.VMEM((16, ROW), jnp.float32),
            pltpu.VMEM((16, ROW), jnp.float32),
            pltpu.VMEM((16, ROW), jnp.float32),
            pltpu.VMEM((16, 128), jnp.float32),
            pltpu.VMEM((128,), jnp.float32),
            pltpu.SemaphoreType.DMA,
            pltpu.SemaphoreType.DMA,
            pltpu.SemaphoreType.DMA,
            pltpu.SemaphoreType.DMA,
        ],
    )
    def k(h_hbm, asrc_hbm, adst_hbm, nn_hbm, bias_hbm, out_hbm,
          acc, nn_v, adt_v, ast_v, h_v, msg_a, msg_b, msg_c, msg_d,
          out_v, bias_v, sem_a, sem_b, sem_c, sem_d):
        c = lax.axis_index("c")
        s = lax.axis_index("s")
        w0 = s * SPW
        iota = lax.iota(jnp.int32, 16)
        zeros16 = jnp.zeros((16,), jnp.float32)

        for gi in range(2):
            g = 2 * c + gi
            col0 = g * 128
            pltpu.sync_copy(adst_hbm.at[pl.ds(2 * g * NPAD, N)], adt_v.at[pl.ds(0, N)])
            pltpu.sync_copy(adst_hbm.at[pl.ds((2 * g + 1) * NPAD, N)],
                            adt_v.at[pl.ds(N, N)])
            for hl in range(2):
                pltpu.sync_copy(
                    asrc_hbm.at[pl.ds((2 * g + hl) * NPAD + w0, SPW)],
                    ast_v.at[hl])
            pltpu.sync_copy(bias_hbm.at[pl.ds(col0, 128)], bias_v)

            def zmsg(i, _):
                for u in range(ROW // 16):
                    msg_a[i, pl.ds(16 * u, 16)] = zeros16
                return 0
            lax.fori_loop(0, 16, zmsg, 0)

            def zero_blk(b, _):
                j0 = w0 + 16 * b
                @pl.when(j0 < N)
                def _():
                    pltpu.sync_copy(msg_a.at[pl.ds(0, 16)],
                                    acc.at[pl.ds(j0, 16)])
                return 0
            lax.fori_loop(0, NBLK, zero_blk, 0)
            plsc.subcore_barrier()

            lane0 = (iota == 0).astype(jnp.float32)
            lane1 = (iota == 1).astype(jnp.float32)

            def emit_source(jj, msg, sem, first, as0, as1):
                d_v = nn_v[pl.ds(16 * jj, 16)]
                @pl.when(jnp.logical_not(first))
                def _():
                    pltpu.make_async_copy(msg, acc.at[d_v], sem).wait()
                ad0 = plsc.load_gather(adt_v, [d_v])
                ad1 = plsc.load_gather(adt_v, [d_v + N])
                w0h = _lrelu_exp(_splat(as0, jj) + ad0)
                w1h = _lrelu_exp(_splat(as1, jj) + ad1)
                hvs = [h_v[jj, pl.ds(16 * v, 16)] for v in range(8)]
                for ke in range(16):
                    ws0 = _splat(w0h, ke)
                    ws1 = _splat(w1h, ke)
                    for v in range(8):
                        msg[ke, pl.ds(16 * v, 16)] = (
                            hvs[v] * (ws0 if v < 4 else ws1))
                    msg[ke, pl.ds(128, 16)] = ws0 * lane0 + ws1 * lane1
                pltpu.async_copy(msg, acc.at[d_v], sem, add=True)

            bufs = [(msg_a, sem_a), (msg_b, sem_b), (msg_c, sem_c),
                    (msg_d, sem_d)]

            def blk(b, _):
                j0 = w0 + 16 * b
                @pl.when(j0 < N)
                def _():
                    pltpu.sync_copy(
                        h_hbm.at[pl.ds(j0, 16), pl.ds(col0, 128)], h_v)
                    pltpu.sync_copy(nn_hbm.at[pl.ds(16 * j0, 256)], nn_v)
                    as0 = ast_v[0, pl.ds(16 * b, 16)]
                    as1 = ast_v[1, pl.ds(16 * b, 16)]
                    for jj in range(16):
                        m, sm = bufs[jj % 4]
                        first = (b == 0) if jj < 4 else False
                        emit_source(jj, m, sm, first, as0, as1)
                return 0
            lax.fori_loop(0, NBLK, blk, 0)
            for m, sm in bufs:
                pltpu.make_async_copy(
                    m, acc.at[nn_v[pl.ds(0, 16)]], sm).wait()
            plsc.subcore_barrier()

            def ep_blk(b, _):
                j0 = w0 + 16 * b
                @pl.when(j0 < N)
                def _():
                    pltpu.sync_copy(acc.at[pl.ds(j0, 16)],
                                    msg_a.at[pl.ds(0, 16)])
                    pltpu.sync_copy(
                        h_hbm.at[pl.ds(j0, 16), pl.ds(col0, 128)], h_v)
                    wl0 = _lrelu_exp(ast_v[0, pl.ds(16 * b, 16)]
                                     + adt_v[pl.ds(j0, 16)])
                    wl1 = _lrelu_exp(ast_v[1, pl.ds(16 * b, 16)]
                                     + adt_v[pl.ds(N + j0, 16)])
                    bvs = [bias_v[pl.ds(16 * v, 16)] for v in range(8)]

                    def node(kk, _):
                        wls0 = _splat(wl0, kk)
                        wls1 = _splat(wl1, kk)
                        dvec = msg_a[kk, pl.ds(128, 16)]
                        dns0 = _splat(dvec, 0) + wls0 + 1e-16
                        dns1 = _splat(dvec, 1) + wls1 + 1e-16
                        for v in range(8):
                            wls, dns = (wls0, dns0) if v < 4 else (wls1, dns1)
                            numv = (msg_a[kk, pl.ds(16 * v, 16)]
                                    + h_v[kk, pl.ds(16 * v, 16)] * wls)
                            out_v[kk, pl.ds(16 * v, 16)] = jnp.maximum(
                                numv / dns + bvs[v], 0.0)
                        return 0
                    lax.fori_loop(0, 16, node, 0)
                    pltpu.sync_copy(
                        out_v, out_hbm.at[pl.ds(j0, 16), pl.ds(col0, 128)])
                return 0
            lax.fori_loop(0, NBLK, ep_blk, 0)
            plsc.subcore_barrier()

    return k(h_p, asrcT_flat, adstT_flat, nn_flat, bias)


def _mm_a_body(x_ref, w_ref, att_ref, h_ref, a_ref):
    x = x_ref[...]
    w = w_ref[...]
    h = lax.dot_general(x, w, (((0,), (0,)), ((), ())),
                        preferred_element_type=jnp.float32,
                        precision=lax.Precision.HIGHEST)
    h_ref[...] = h
    a_ref[...] = lax.dot_general(h, att_ref[...], (((1,), (0,)), ((), ())),
                                 preferred_element_type=jnp.float32,
                                 precision=lax.Precision.HIGHEST)


def _matmul_a(x2d, W_gat, att_mat, bn=1024):
    f, n = x2d.shape
    c = W_gat.shape[1]
    return pl.pallas_call(
        _mm_a_body,
        grid=(n // bn,),
        in_specs=[
            pl.BlockSpec((f, bn), lambda i: (0, i)),
            pl.BlockSpec((f, c), lambda i: (0, 0)),
            pl.BlockSpec((c, 16), lambda i: (0, 0)),
        ],
        out_specs=[
            pl.BlockSpec((bn, c), lambda i: (i, 0)),
            pl.BlockSpec((bn, 16), lambda i: (i, 0)),
        ],
        out_shape=[
            jax.ShapeDtypeStruct((n, c), jnp.float32),
            jax.ShapeDtypeStruct((n, 16), jnp.float32),
        ],
    )(x2d, W_gat, att_mat)


def _mm_c_body(v_ref, w_ref, b_ref, y_ref):
    v = v_ref[...]
    w = w_ref[...]
    y = lax.dot_general(v, w, (((0,), (1,)), ((), ())),
                        preferred_element_type=jnp.float32,
                        precision=lax.Precision.HIGHEST)
    y_ref[...] = y + b_ref[...]


def _matmul_c(V, fc_W, fc_b, bn=1024):
    c, n = V.shape
    return pl.pallas_call(
        _mm_c_body,
        grid=(n // bn,),
        in_specs=[
            pl.BlockSpec((c, bn), lambda i: (0, i)),
            pl.BlockSpec(fc_W.shape, lambda i: (0, 0)),
            pl.BlockSpec((1, c), lambda i: (0, 0)),
        ],
        out_specs=pl.BlockSpec((bn, c), lambda i: (i, 0)),
        out_shape=jax.ShapeDtypeStruct((n, c), jnp.float32),
    )(V, fc_W, fc_b.reshape(1, -1))


def kernel(x, edge_index, W_gat, att_src, att_dst, bias_gat, fc_W, fc_b):
    B, F_in, N, _ = x.shape
    K = edge_index.shape[-1]
    H = HEADS
    C_out = W_gat.shape[1]
    C = C_out // H
    NP = ((N + 1023) // 1024) * 1024

    x2d = x.reshape(F_in, N)
    x2d_p = jnp.pad(x2d, ((0, 0), (0, NP - N)))
    nn_p = jnp.pad(edge_index[0].reshape(N, K), ((0, NP - N), (0, 0)))

    eye = jnp.eye(H, dtype=jnp.float32)
    m_src = (eye[:, None, :] * att_src[:, :, None]).reshape(C_out, H)
    m_dst = (eye[:, None, :] * att_dst[:, :, None]).reshape(C_out, H)
    att_mat = jnp.concatenate([m_src, m_dst], axis=1)

    h_p, a_p = _matmul_a(x2d_p, W_gat, att_mat)
    asrcT = a_p[:, :H].T.reshape(-1)
    adstT = a_p[:, H:].T.reshape(-1)

    g = _gat_sc(h_p, asrcT, adstT, nn_p.reshape(-1), bias_gat)

    V = g.reshape(N * C_out).reshape(C_out, N)
    V_p = jnp.pad(V, ((0, 0), (0, NP - N)))
    y = _matmul_c(V_p, fc_W, fc_b)[:N]
    s = int(N ** 0.5)
    return y.reshape(B, C_out, s, s)


# speedup vs baseline: 1.3807x; 1.3807x over previous
import functools

import jax
import jax.numpy as jnp
from jax import lax
from jax.experimental import pallas as pl
from jax.experimental.pallas import tpu as pltpu
from jax.experimental.pallas import tpu_sc as plsc

HEADS = 8
NEG_SLOPE = 0.2
_GDN = lax.GatherDimensionNumbers(
    offset_dims=(), collapsed_slice_dims=(0,), start_index_map=(0,))


def _splat(vec, i):
    idx = jnp.full((16, 1), i, dtype=jnp.int32)
    return lax.gather(vec, idx, _GDN, (1,),
                      mode=lax.GatherScatterMode.PROMISE_IN_BOUNDS)


def _lrelu_exp(z):
    return jnp.exp(jnp.maximum(z, NEG_SLOPE * z))


def _gat_sc(h_p, asrcT_flat, adstT_flat, nn_flat, bias):
    NPAD, C_out = h_p.shape
    N = 10000
    ROW = 144
    SPW = NPAD // 16
    NBLK = SPW // 16
    mesh = plsc.VectorSubcoreMesh(core_axis_name="c", subcore_axis_name="s")

    @functools.partial(
        pl.kernel,
        mesh=mesh,
        compiler_params=pltpu.CompilerParams(
            needs_layout_passes=False, use_tc_tiling_on_sc=False),
        out_type=jax.ShapeDtypeStruct((N, C_out), jnp.float32),
        scratch_types=[
            pltpu.VMEM_SHARED((N, ROW), jnp.float32),
            pltpu.VMEM((16 * 640,), jnp.int32),
            pltpu.VMEM((2 * N,), jnp.float32),
            pltpu.VMEM((2, SPW), jnp.float32),
            pltpu.VMEM((16, 128), jnp.float32),
            pltpu.VMEM((16, ROW), jnp.float32),
            pltpu.VMEM((16, ROW), jnp.float32),
            pltpu.VMEM((16, 128), jnp.float32),
            pltpu.VMEM((128,), jnp.float32),
            pltpu.SemaphoreType.DMA,
            pltpu.SemaphoreType.DMA,
        ],
    )
    def k(h_hbm, asrc_hbm, adst_hbm, nn_hbm, bias_hbm, out_hbm,
          acc, nn_v, adt_v, ast_v, h_v, msg_a, msg_b,
          out_v, bias_v, sem_a, sem_b):
        c = lax.axis_index("c")
        s = lax.axis_index("s")
        w0 = s * SPW
        iota = lax.iota(jnp.int32, 16)
        zeros16 = jnp.zeros((16,), jnp.float32)

        pltpu.sync_copy(nn_hbm.at[pl.ds(16 * w0, 16 * SPW)], nn_v)

        for gi in range(2):
            g = 2 * c + gi
            col0 = g * 128
            pltpu.sync_copy(adst_hbm.at[pl.ds(2 * g * NPAD, N)], adt_v.at[pl.ds(0, N)])
            pltpu.sync_copy(adst_hbm.at[pl.ds((2 * g + 1) * NPAD, N)],
                            adt_v.at[pl.ds(N, N)])
            for hl in range(2):
                pltpu.sync_copy(
                    asrc_hbm.at[pl.ds((2 * g + hl) * NPAD + w0, SPW)],
                    ast_v.at[hl])
            pltpu.sync_copy(bias_hbm.at[pl.ds(col0, 128)], bias_v)

            def zmsg(i, _):
                for u in range(ROW // 16):
                    msg_a[i, pl.ds(16 * u, 16)] = zeros16
                return 0
            lax.fori_loop(0, 16, zmsg, 0)

            def zero_blk(b, _):
                j0 = w0 + 16 * b
                @pl.when(j0 < N)
                def _():
                    pltpu.sync_copy(msg_a.at[pl.ds(0, 16)],
                                    acc.at[pl.ds(j0, 16)])
                return 0
            lax.fori_loop(0, NBLK, zero_blk, 0)
            plsc.subcore_barrier()

            lane0 = (iota == 0).astype(jnp.float32)
            lane1 = (iota == 1).astype(jnp.float32)

            def emit_source(b, jj, msg, sem, first, as0, as1):
                jl = 16 * b + jj
                d_v = nn_v[pl.ds(16 * jl, 16)]
                @pl.when(jnp.logical_not(first))
                def _():
                    pltpu.make_async_copy(msg, acc.at[d_v], sem).wait()
                ad0 = plsc.load_gather(adt_v, [d_v])
                ad1 = plsc.load_gather(adt_v, [d_v + N])
                w0h = _lrelu_exp(_splat(as0, jj) + ad0)
                w1h = _lrelu_exp(_splat(as1, jj) + ad1)
                hvs = [h_v[jj, pl.ds(16 * v, 16)] for v in range(8)]
                for ke in range(16):
                    ws0 = _splat(w0h, ke)
                    ws1 = _splat(w1h, ke)
                    for v in range(8):
                        msg[ke, pl.ds(16 * v, 16)] = (
                            hvs[v] * (ws0 if v < 4 else ws1))
                    msg[ke, pl.ds(128, 16)] = ws0 * lane0 + ws1 * lane1
                pltpu.async_copy(msg, acc.at[d_v], sem, add=True)

            def blk(b, _):
                j0 = w0 + 16 * b
                @pl.when(j0 < N)
                def _():
                    pltpu.sync_copy(
                        h_hbm.at[pl.ds(j0, 16), pl.ds(col0, 128)], h_v)
                    as0 = ast_v[0, pl.ds(16 * b, 16)]
                    as1 = ast_v[1, pl.ds(16 * b, 16)]

                    def pair(jp, _):
                        first = jnp.logical_and(b == 0, jp == 0)
                        emit_source(b, 2 * jp, msg_a, sem_a, first, as0, as1)
                        emit_source(b, 2 * jp + 1, msg_b, sem_b, first,
                                    as0, as1)
                        return 0
                    lax.fori_loop(0, 8, pair, 0)
                return 0
            lax.fori_loop(0, NBLK, blk, 0)
            for m, sm in ((msg_a, sem_a), (msg_b, sem_b)):
                pltpu.make_async_copy(
                    m, acc.at[nn_v[pl.ds(0, 16)]], sm).wait()
            plsc.subcore_barrier()

            def ep_blk(b, _):
                j0 = w0 + 16 * b
                @pl.when(j0 < N)
                def _():
                    pltpu.sync_copy(acc.at[pl.ds(j0, 16)],
                                    msg_a.at[pl.ds(0, 16)])
                    pltpu.sync_copy(
                        h_hbm.at[pl.ds(j0, 16), pl.ds(col0, 128)], h_v)
                    wl0 = _lrelu_exp(ast_v[0, pl.ds(16 * b, 16)]
                                     + adt_v[pl.ds(j0, 16)])
                    wl1 = _lrelu_exp(ast_v[1, pl.ds(16 * b, 16)]
                                     + adt_v[pl.ds(N + j0, 16)])
                    bvs = [bias_v[pl.ds(16 * v, 16)] for v in range(8)]

                    def node(kk, _):
                        wls0 = _splat(wl0, kk)
                        wls1 = _splat(wl1, kk)
                        dvec = msg_a[kk, pl.ds(128, 16)]
                        dns0 = _splat(dvec, 0) + wls0 + 1e-16
                        dns1 = _splat(dvec, 1) + wls1 + 1e-16
                        for v in range(8):
                            wls, dns = (wls0, dns0) if v < 4 else (wls1, dns1)
                            numv = (msg_a[kk, pl.ds(16 * v, 16)]
                                    + h_v[kk, pl.ds(16 * v, 16)] * wls)
                            out_v[kk, pl.ds(16 * v, 16)] = jnp.maximum(
                                numv / dns + bvs[v], 0.0)
                        return 0
                    lax.fori_loop(0, 16, node, 0)
                    pltpu.sync_copy(
                        out_v, out_hbm.at[pl.ds(j0, 16), pl.ds(col0, 128)])
                return 0
            lax.fori_loop(0, NBLK, ep_blk, 0)
            plsc.subcore_barrier()

    return k(h_p, asrcT_flat, adstT_flat, nn_flat, bias)


def _mm_a_body(x_ref, w_ref, att_ref, h_ref, a_ref):
    x = x_ref[...]
    w = w_ref[...]
    h = lax.dot_general(x, w, (((0,), (0,)), ((), ())),
                        preferred_element_type=jnp.float32,
                        precision=lax.Precision.HIGHEST)
    h_ref[...] = h
    a_ref[...] = lax.dot_general(h, att_ref[...], (((1,), (0,)), ((), ())),
                                 preferred_element_type=jnp.float32,
                                 precision=lax.Precision.HIGHEST)


def _matmul_a(x2d, W_gat, att_mat, bn=1024):
    f, n = x2d.shape
    c = W_gat.shape[1]
    return pl.pallas_call(
        _mm_a_body,
        grid=(n // bn,),
        in_specs=[
            pl.BlockSpec((f, bn), lambda i: (0, i)),
            pl.BlockSpec((f, c), lambda i: (0, 0)),
            pl.BlockSpec((c, 16), lambda i: (0, 0)),
        ],
        out_specs=[
            pl.BlockSpec((bn, c), lambda i: (i, 0)),
            pl.BlockSpec((bn, 16), lambda i: (i, 0)),
        ],
        out_shape=[
            jax.ShapeDtypeStruct((n, c), jnp.float32),
            jax.ShapeDtypeStruct((n, 16), jnp.float32),
        ],
    )(x2d, W_gat, att_mat)


def _mm_c_body(v_ref, w_ref, b_ref, y_ref):
    v = v_ref[...]
    w = w_ref[...]
    y = lax.dot_general(v, w, (((0,), (1,)), ((), ())),
                        preferred_element_type=jnp.float32,
                        precision=lax.Precision.HIGHEST)
    y_ref[...] = y + b_ref[...]


def _matmul_c(V, fc_W, fc_b, bn=1024):
    c, n = V.shape
    return pl.pallas_call(
        _mm_c_body,
        grid=(n // bn,),
        in_specs=[
            pl.BlockSpec((c, bn), lambda i: (0, i)),
            pl.BlockSpec(fc_W.shape, lambda i: (0, 0)),
            pl.BlockSpec((1, c), lambda i: (0, 0)),
        ],
        out_specs=pl.BlockSpec((bn, c), lambda i: (i, 0)),
        out_shape=jax.ShapeDtypeStruct((n, c), jnp.float32),
    )(V, fc_W, fc_b.reshape(1, -1))


def kernel(x, edge_index, W_gat, att_src, att_dst, bias_gat, fc_W, fc_b):
    B, F_in, N, _ = x.shape
    K = edge_index.shape[-1]
    H = HEADS
    C_out = W_gat.shape[1]
    C = C_out // H
    NP = ((N + 1023) // 1024) * 1024

    x2d = x.reshape(F_in, N)
    x2d_p = jnp.pad(x2d, ((0, 0), (0, NP - N)))
    nn_p = jnp.pad(edge_index[0].reshape(N, K), ((0, NP - N), (0, 0)))

    eye = jnp.eye(H, dtype=jnp.float32)
    m_src = (eye[:, None, :] * att_src[:, :, None]).reshape(C_out, H)
    m_dst = (eye[:, None, :] * att_dst[:, :, None]).reshape(C_out, H)
    att_mat = jnp.concatenate([m_src, m_dst], axis=1)

    h_p, a_p = _matmul_a(x2d_p, W_gat, att_mat)
    asrcT = a_p[:, :H].T.reshape(-1)
    adstT = a_p[:, H:].T.reshape(-1)

    g = _gat_sc(h_p, asrcT, adstT, nn_p.reshape(-1), bias_gat)

    V = g.reshape(N * C_out).reshape(C_out, N)
    V_p = jnp.pad(V, ((0, 0), (0, NP - N)))
    y = _matmul_c(V_p, fc_W, fc_b)[:N]
    s = int(N ** 0.5)
    return y.reshape(B, C_out, s, s)


# speedup vs baseline: 1.4280x; 1.0343x over previous
import functools

import jax
import jax.numpy as jnp
from jax import lax
from jax.experimental import pallas as pl
from jax.experimental.pallas import tpu as pltpu
from jax.experimental.pallas import tpu_sc as plsc

HEADS = 8
NEG_SLOPE = 0.2
_GDN = lax.GatherDimensionNumbers(
    offset_dims=(), collapsed_slice_dims=(0,), start_index_map=(0,))


def _splat(vec, i):
    idx = jnp.full((16, 1), i, dtype=jnp.int32)
    return lax.gather(vec, idx, _GDN, (1,),
                      mode=lax.GatherScatterMode.PROMISE_IN_BOUNDS)


def _lrelu_exp(z):
    return jnp.exp(jnp.maximum(z, NEG_SLOPE * z))


def _gat_sc(h_p, asrcT_flat, adstT_flat, nn_flat, bias):
    NPAD, C_out = h_p.shape
    N = 10000
    ROW = 144
    SPW = NPAD // 16
    NBLK = SPW // 16
    mesh = plsc.VectorSubcoreMesh(core_axis_name="c", subcore_axis_name="s")

    @functools.partial(
        pl.kernel,
        mesh=mesh,
        compiler_params=pltpu.CompilerParams(
            needs_layout_passes=False, use_tc_tiling_on_sc=False),
        out_type=jax.ShapeDtypeStruct((N, C_out), jnp.float32),
        scratch_types=[
            pltpu.VMEM_SHARED((N, ROW), jnp.float32),
            pltpu.VMEM((16 * 640,), jnp.int32),
            pltpu.VMEM((2 * N,), jnp.float32),
            pltpu.VMEM((2, SPW), jnp.float32),
            pltpu.VMEM((16, 128), jnp.float32),
            pltpu.VMEM((16, ROW), jnp.float32),
            pltpu.VMEM((16, ROW), jnp.float32),
            pltpu.VMEM((16, 128), jnp.float32),
            pltpu.VMEM((128,), jnp.float32),
            pltpu.SemaphoreType.DMA,
            pltpu.SemaphoreType.DMA,
            pltpu.SemaphoreType.DMA,
        ],
    )
    def k(h_hbm, asrc_hbm, adst_hbm, nn_hbm, bias_hbm, out_hbm,
          acc, nn_v, adt_v, ast_v, h_v, msg_a, msg_b,
          out_v, bias_v, sem_a, sem_b, sem_o):
        c = lax.axis_index("c")
        s = lax.axis_index("s")
        w0 = s * SPW
        iota = lax.iota(jnp.int32, 16)
        zeros16 = jnp.zeros((16,), jnp.float32)

        pltpu.sync_copy(nn_hbm.at[pl.ds(16 * w0, 16 * SPW)], nn_v)

        for gi in range(2):
            g = 2 * c + gi
            col0 = g * 128
            pltpu.sync_copy(adst_hbm.at[pl.ds(2 * g * NPAD, N)], adt_v.at[pl.ds(0, N)])
            pltpu.sync_copy(adst_hbm.at[pl.ds((2 * g + 1) * NPAD, N)],
                            adt_v.at[pl.ds(N, N)])
            for hl in range(2):
                pltpu.sync_copy(
                    asrc_hbm.at[pl.ds((2 * g + hl) * NPAD + w0, SPW)],
                    ast_v.at[hl])
            pltpu.sync_copy(bias_hbm.at[pl.ds(col0, 128)], bias_v)

            def zmsg(i, _):
                for u in range(ROW // 16):
                    msg_a[i, pl.ds(16 * u, 16)] = zeros16
                return 0
            lax.fori_loop(0, 16, zmsg, 0)

            def zero_blk(b, _):
                j0 = w0 + 16 * b
                @pl.when(j0 < N)
                def _():
                    pltpu.sync_copy(msg_a.at[pl.ds(0, 16)],
                                    acc.at[pl.ds(j0, 16)])
                return 0
            lax.fori_loop(0, NBLK, zero_blk, 0)
            plsc.subcore_barrier()

            lane0 = (iota == 0).astype(jnp.float32)
            lane1 = (iota == 1).astype(jnp.float32)

            def emit_source(b, jj, msg, sem, first, as0, as1):
                jl = 16 * b + jj
                d_v = nn_v[pl.ds(16 * jl, 16)]
                @pl.when(jnp.logical_not(first))
                def _():
                    pltpu.make_async_copy(msg, acc.at[d_v], sem).wait()
                ad0 = plsc.load_gather(adt_v, [d_v])
                ad1 = plsc.load_gather(adt_v, [d_v + N])
                w0h = _lrelu_exp(_splat(as0, jj) + ad0)
                w1h = _lrelu_exp(_splat(as1, jj) + ad1)
                hvs = [h_v[jj, pl.ds(16 * v, 16)] for v in range(8)]
                for ke in range(16):
                    ws0 = _splat(w0h, ke)
                    ws1 = _splat(w1h, ke)
                    for v in range(8):
                        msg[ke, pl.ds(16 * v, 16)] = (
                            hvs[v] * (ws0 if v < 4 else ws1))
                    msg[ke, pl.ds(128, 16)] = ws0 * lane0 + ws1 * lane1
                pltpu.async_copy(msg, acc.at[d_v], sem, add=True)

            def blk(b, _):
                j0 = w0 + 16 * b
                @pl.when(j0 < N)
                def _():
                    pltpu.sync_copy(
                        h_hbm.at[pl.ds(j0, 16), pl.ds(col0, 128)], h_v)
                    as0 = ast_v[0, pl.ds(16 * b, 16)]
                    as1 = ast_v[1, pl.ds(16 * b, 16)]

                    def pair(jp, _):
                        first = jnp.logical_and(b == 0, jp == 0)
                        emit_source(b, 2 * jp, msg_a, sem_a, first, as0, as1)
                        emit_source(b, 2 * jp + 1, msg_b, sem_b, first,
                                    as0, as1)
                        return 0
                    lax.fori_loop(0, 8, pair, 0)
                return 0
            lax.fori_loop(0, NBLK, blk, 0)
            for m, sm in ((msg_a, sem_a), (msg_b, sem_b)):
                pltpu.make_async_copy(
                    m, acc.at[nn_v[pl.ds(0, 16)]], sm).wait()
            plsc.subcore_barrier()

            def ep_blk(b, _):
                j0 = w0 + 16 * b
                @pl.when(j0 < N)
                def _():
                    rd_acc = pltpu.async_copy(acc.at[pl.ds(j0, 16)],
                                              msg_a.at[pl.ds(0, 16)], sem_a)
                    rd_h = pltpu.async_copy(
                        h_hbm.at[pl.ds(j0, 16), pl.ds(col0, 128)], h_v, sem_b)
                    @pl.when(jnp.logical_or(b > 0, gi > 0))
                    def _():
                        pltpu.make_async_copy(
                            out_v,
                            out_hbm.at[pl.ds(j0, 16), pl.ds(col0, 128)],
                            sem_o).wait()
                    rd_acc.wait()
                    rd_h.wait()
                    wl0 = _lrelu_exp(ast_v[0, pl.ds(16 * b, 16)]
                                     + adt_v[pl.ds(j0, 16)])
                    wl1 = _lrelu_exp(ast_v[1, pl.ds(16 * b, 16)]
                                     + adt_v[pl.ds(N + j0, 16)])
                    bvs = [bias_v[pl.ds(16 * v, 16)] for v in range(8)]

                    def node(kk, _):
                        wls0 = _splat(wl0, kk)
                        wls1 = _splat(wl1, kk)
                        dvec = msg_a[kk, pl.ds(128, 16)]
                        dns0 = _splat(dvec, 0) + wls0 + 1e-16
                        dns1 = _splat(dvec, 1) + wls1 + 1e-16
                        for v in range(8):
                            wls, dns = (wls0, dns0) if v < 4 else (wls1, dns1)
                            numv = (msg_a[kk, pl.ds(16 * v, 16)]
                                    + h_v[kk, pl.ds(16 * v, 16)] * wls)
                            out_v[kk, pl.ds(16 * v, 16)] = jnp.maximum(
                                numv / dns + bvs[v], 0.0)
                        return 0
                    lax.fori_loop(0, 16, node, 0)
                    pltpu.async_copy(
                        out_v, out_hbm.at[pl.ds(j0, 16), pl.ds(col0, 128)],
                        sem_o)
                return 0
            lax.fori_loop(0, NBLK, ep_blk, 0)
            plsc.subcore_barrier()

        pltpu.make_async_copy(
            out_v, out_hbm.at[pl.ds(w0, 16), pl.ds(0, 128)], sem_o).wait()

    return k(h_p, asrcT_flat, adstT_flat, nn_flat, bias)


def _mm_a_body(x_ref, w_ref, att_ref, h_ref, a_ref):
    x = x_ref[...]
    w = w_ref[...]
    h = lax.dot_general(x, w, (((0,), (0,)), ((), ())),
                        preferred_element_type=jnp.float32,
                        precision=lax.Precision.HIGHEST)
    h_ref[...] = h
    a_ref[...] = lax.dot_general(h, att_ref[...], (((1,), (0,)), ((), ())),
                                 preferred_element_type=jnp.float32,
                                 precision=lax.Precision.HIGHEST)


def _matmul_a(x2d, W_gat, att_mat, bn=1024):
    f, n = x2d.shape
    c = W_gat.shape[1]
    return pl.pallas_call(
        _mm_a_body,
        grid=(n // bn,),
        in_specs=[
            pl.BlockSpec((f, bn), lambda i: (0, i)),
            pl.BlockSpec((f, c), lambda i: (0, 0)),
            pl.BlockSpec((c, 16), lambda i: (0, 0)),
        ],
        out_specs=[
            pl.BlockSpec((bn, c), lambda i: (i, 0)),
            pl.BlockSpec((bn, 16), lambda i: (i, 0)),
        ],
        out_shape=[
            jax.ShapeDtypeStruct((n, c), jnp.float32),
            jax.ShapeDtypeStruct((n, 16), jnp.float32),
        ],
    )(x2d, W_gat, att_mat)


def _mm_c_body(v_ref, w_ref, b_ref, y_ref):
    v = v_ref[...]
    w = w_ref[...]
    y = lax.dot_general(v, w, (((0,), (1,)), ((), ())),
                        preferred_element_type=jnp.float32,
                        precision=lax.Precision.HIGHEST)
    y_ref[...] = y + b_ref[...]


def _matmul_c(V, fc_W, fc_b, bn=1024):
    c, n = V.shape
    return pl.pallas_call(
        _mm_c_body,
        grid=(n // bn,),
        in_specs=[
            pl.BlockSpec((c, bn), lambda i: (0, i)),
            pl.BlockSpec(fc_W.shape, lambda i: (0, 0)),
            pl.BlockSpec((1, c), lambda i: (0, 0)),
        ],
        out_specs=pl.BlockSpec((bn, c), lambda i: (i, 0)),
        out_shape=jax.ShapeDtypeStruct((n, c), jnp.float32),
    )(V, fc_W, fc_b.reshape(1, -1))


def kernel(x, edge_index, W_gat, att_src, att_dst, bias_gat, fc_W, fc_b):
    B, F_in, N, _ = x.shape
    K = edge_index.shape[-1]
    H = HEADS
    C_out = W_gat.shape[1]
    C = C_out // H
    NP = ((N + 1023) // 1024) * 1024

    x2d = x.reshape(F_in, N)
    x2d_p = jnp.pad(x2d, ((0, 0), (0, NP - N)))
    nn_p = jnp.pad(edge_index[0].reshape(N, K), ((0, NP - N), (0, 0)))

    eye = jnp.eye(H, dtype=jnp.float32)
    m_src = (eye[:, None, :] * att_src[:, :, None]).reshape(C_out, H)
    m_dst = (eye[:, None, :] * att_dst[:, :, None]).reshape(C_out, H)
    att_mat = jnp.concatenate([m_src, m_dst], axis=1)

    h_p, a_p = _matmul_a(x2d_p, W_gat, att_mat)
    asrcT = a_p[:, :H].T.reshape(-1)
    adstT = a_p[:, H:].T.reshape(-1)

    g = _gat_sc(h_p, asrcT, adstT, nn_p.reshape(-1), bias_gat)

    V = g.reshape(N * C_out).reshape(C_out, N)
    V_p = jnp.pad(V, ((0, 0), (0, NP - N)))
    y = _matmul_c(V_p, fc_W, fc_b)[:N]
    s = int(N ** 0.5)
    return y.reshape(B, C_out, s, s)


# speedup vs baseline: 1.5398x; 1.0783x over previous
import functools

import jax
import jax.numpy as jnp
from jax import lax
from jax.experimental import pallas as pl
from jax.experimental.pallas import tpu as pltpu
from jax.experimental.pallas import tpu_sc as plsc

HEADS = 8
NEG_SLOPE = 0.2
_GDN = lax.GatherDimensionNumbers(
    offset_dims=(), collapsed_slice_dims=(0,), start_index_map=(0,))


def _splat(vec, i):
    idx = jnp.full((16, 1), i, dtype=jnp.int32)
    return lax.gather(vec, idx, _GDN, (1,),
                      mode=lax.GatherScatterMode.PROMISE_IN_BOUNDS)


def _lrelu_exp(z):
    return jnp.exp(jnp.maximum(z, NEG_SLOPE * z))


def _gat_sc(h_p, asrcT_flat, adstT_flat, nn_flat, bias):
    NPAD, C_out = h_p.shape
    N = 10000
    ROW = 144
    SPW = NPAD // 16
    NBLK = SPW // 16
    mesh = plsc.VectorSubcoreMesh(core_axis_name="c", subcore_axis_name="s")

    @functools.partial(
        pl.kernel,
        mesh=mesh,
        compiler_params=pltpu.CompilerParams(
            needs_layout_passes=False, use_tc_tiling_on_sc=False),
        out_type=jax.ShapeDtypeStruct((N, C_out), jnp.float32),
        scratch_types=[
            pltpu.VMEM_SHARED((N, ROW), jnp.float32),
            pltpu.VMEM((16 * 640,), jnp.int32),
            pltpu.VMEM((2 * N,), jnp.float32),
            pltpu.VMEM((2, SPW), jnp.float32),
            pltpu.VMEM((16, 128), jnp.float32),
            pltpu.VMEM((16, ROW), jnp.float32),
            pltpu.VMEM((16, ROW), jnp.float32),
            pltpu.VMEM((16, 128), jnp.float32),
            pltpu.VMEM((128,), jnp.float32),
            pltpu.SemaphoreType.DMA,
            pltpu.SemaphoreType.DMA,
            pltpu.SemaphoreType.DMA,
            pltpu.SemaphoreType.DMA,
            pltpu.SemaphoreType.DMA,
        ],
    )
    def k(h_hbm, asrc_hbm, adst_hbm, nn_hbm, bias_hbm, out_hbm,
          acc, nn_v, adt_v, ast_v, h_a, msg_a, msg_b,
          h_b, bias_v, sem_a, sem_b, sem_o, sem_ha, sem_hb):
        c = lax.axis_index("c")
        s = lax.axis_index("s")
        w0 = s * SPW
        iota = lax.iota(jnp.int32, 16)
        zeros16 = jnp.zeros((16,), jnp.float32)

        pltpu.sync_copy(nn_hbm.at[pl.ds(16 * w0, 16 * SPW)], nn_v)

        for gi in range(2):
            g = 2 * c + gi
            col0 = g * 128
            pltpu.sync_copy(adst_hbm.at[pl.ds(2 * g * NPAD, N)], adt_v.at[pl.ds(0, N)])
            pltpu.sync_copy(adst_hbm.at[pl.ds((2 * g + 1) * NPAD, N)],
                            adt_v.at[pl.ds(N, N)])
            for hl in range(2):
                pltpu.sync_copy(
                    asrc_hbm.at[pl.ds((2 * g + hl) * NPAD + w0, SPW)],
                    ast_v.at[hl])
            pltpu.sync_copy(bias_hbm.at[pl.ds(col0, 128)], bias_v)

            def zmsg(i, _):
                for u in range(ROW // 16):
                    msg_a[i, pl.ds(16 * u, 16)] = zeros16
                return 0
            lax.fori_loop(0, 16, zmsg, 0)

            def zero_blk(b, _):
                j0 = w0 + 16 * b
                @pl.when(j0 < N)
                def _():
                    pltpu.sync_copy(msg_a.at[pl.ds(0, 16)],
                                    acc.at[pl.ds(j0, 16)])
                return 0
            lax.fori_loop(0, NBLK, zero_blk, 0)
            plsc.subcore_barrier()

            lane0 = (iota == 0).astype(jnp.float32)
            lane1 = (iota == 1).astype(jnp.float32)

            def emit_source(b, jj, hbuf, msg, sem, first, as0, as1):
                jl = 16 * b + jj
                d_v = nn_v[pl.ds(16 * jl, 16)]
                @pl.when(jnp.logical_not(first))
                def _():
                    pltpu.make_async_copy(msg, acc.at[d_v], sem).wait()
                ad0 = plsc.load_gather(adt_v, [d_v])
                ad1 = plsc.load_gather(adt_v, [d_v + N])
                w0h = _lrelu_exp(_splat(as0, jj) + ad0)
                w1h = _lrelu_exp(_splat(as1, jj) + ad1)
                hvs = [hbuf[jj, pl.ds(16 * v, 16)] for v in range(8)]
                for ke in range(16):
                    ws0 = _splat(w0h, ke)
                    ws1 = _splat(w1h, ke)
                    for v in range(8):
                        msg[ke, pl.ds(16 * v, 16)] = (
                            hvs[v] * (ws0 if v < 4 else ws1))
                    msg[ke, pl.ds(128, 16)] = ws0 * lane0 + ws1 * lane1
                pltpu.async_copy(msg, acc.at[d_v], sem, add=True)

            def h_desc(b, buf, sem):
                j0 = w0 + 16 * b
                return pltpu.make_async_copy(
                    h_hbm.at[pl.ds(j0, 16), pl.ds(col0, 128)], buf, sem)

            def h_fire(b, buf, sem):
                j0 = w0 + 16 * b
                @pl.when(jnp.logical_and(b < NBLK, j0 < N))
                def _():
                    h_desc(b, buf, sem).start()

            def h_wait(b, buf, sem):
                j0 = w0 + 16 * b
                @pl.when(jnp.logical_and(b < NBLK, j0 < N))
                def _():
                    h_desc(b, buf, sem).wait()

            def proc_blk(b, hbuf, _first_ok):
                j0 = w0 + 16 * b
                @pl.when(j0 < N)
                def _():
                    as0 = ast_v[0, pl.ds(16 * b, 16)]
                    as1 = ast_v[1, pl.ds(16 * b, 16)]

                    def pair(jp, _):
                        first = jnp.logical_and(_first_ok, jp == 0)
                        emit_source(b, 2 * jp, hbuf, msg_a, sem_a, first,
                                    as0, as1)
                        emit_source(b, 2 * jp + 1, hbuf, msg_b, sem_b, first,
                                    as0, as1)
                        return 0
                    lax.fori_loop(0, 8, pair, 0)

            h_fire(0, h_a, sem_ha)

            def bp_loop(bp, _):
                b0 = 2 * bp
                h_fire(b0 + 1, h_b, sem_hb)
                h_wait(b0, h_a, sem_ha)
                proc_blk(b0, h_a, bp == 0)
                h_fire(b0 + 2, h_a, sem_ha)
                h_wait(b0 + 1, h_b, sem_hb)
                proc_blk(b0 + 1, h_b, False)
                return 0
            lax.fori_loop(0, NBLK // 2, bp_loop, 0)
            for m, sm in ((msg_a, sem_a), (msg_b, sem_b)):
                pltpu.make_async_copy(
                    m, acc.at[nn_v[pl.ds(0, 16)]], sm).wait()
            plsc.subcore_barrier()

            def ep_blk(b, _):
                j0 = w0 + 16 * b
                @pl.when(j0 < N)
                def _():
                    rd_acc = pltpu.async_copy(acc.at[pl.ds(j0, 16)],
                                              msg_a.at[pl.ds(0, 16)], sem_a)
                    rd_h = pltpu.async_copy(
                        h_hbm.at[pl.ds(j0, 16), pl.ds(col0, 128)], h_a, sem_b)
                    @pl.when(jnp.logical_or(b > 0, gi > 0))
                    def _():
                        pltpu.make_async_copy(
                            h_b,
                            out_hbm.at[pl.ds(j0, 16), pl.ds(col0, 128)],
                            sem_o).wait()
                    rd_acc.wait()
                    rd_h.wait()
                    wl0 = _lrelu_exp(ast_v[0, pl.ds(16 * b, 16)]
                                     + adt_v[pl.ds(j0, 16)])
                    wl1 = _lrelu_exp(ast_v[1, pl.ds(16 * b, 16)]
                                     + adt_v[pl.ds(N + j0, 16)])
                    bvs = [bias_v[pl.ds(16 * v, 16)] for v in range(8)]

                    def node(kk, _):
                        wls0 = _splat(wl0, kk)
                        wls1 = _splat(wl1, kk)
                        dvec = msg_a[kk, pl.ds(128, 16)]
                        dns0 = _splat(dvec, 0) + wls0 + 1e-16
                        dns1 = _splat(dvec, 1) + wls1 + 1e-16
                        for v in range(8):
                            wls, dns = (wls0, dns0) if v < 4 else (wls1, dns1)
                            numv = (msg_a[kk, pl.ds(16 * v, 16)]
                                    + h_a[kk, pl.ds(16 * v, 16)] * wls)
                            h_b[kk, pl.ds(16 * v, 16)] = jnp.maximum(
                                numv / dns + bvs[v], 0.0)
                        return 0
                    lax.fori_loop(0, 16, node, 0)
                    pltpu.async_copy(
                        h_b, out_hbm.at[pl.ds(j0, 16), pl.ds(col0, 128)],
                        sem_o)
                return 0
            lax.fori_loop(0, NBLK, ep_blk, 0)
            plsc.subcore_barrier()

        pltpu.make_async_copy(
            h_b, out_hbm.at[pl.ds(w0, 16), pl.ds(0, 128)], sem_o).wait()

    return k(h_p, asrcT_flat, adstT_flat, nn_flat, bias)


def _mm_a_body(x_ref, w_ref, att_ref, h_ref, a_ref):
    x = x_ref[...]
    w = w_ref[...]
    h = lax.dot_general(x, w, (((0,), (0,)), ((), ())),
                        preferred_element_type=jnp.float32,
                        precision=lax.Precision.HIGHEST)
    h_ref[...] = h
    a_ref[...] = lax.dot_general(h, att_ref[...], (((1,), (0,)), ((), ())),
                                 preferred_element_type=jnp.float32,
                                 precision=lax.Precision.HIGHEST)


def _matmul_a(x2d, W_gat, att_mat, bn=1024):
    f, n = x2d.shape
    c = W_gat.shape[1]
    return pl.pallas_call(
        _mm_a_body,
        grid=(n // bn,),
        in_specs=[
            pl.BlockSpec((f, bn), lambda i: (0, i)),
            pl.BlockSpec((f, c), lambda i: (0, 0)),
            pl.BlockSpec((c, 16), lambda i: (0, 0)),
        ],
        out_specs=[
            pl.BlockSpec((bn, c), lambda i: (i, 0)),
            pl.BlockSpec((bn, 16), lambda i: (i, 0)),
        ],
        out_shape=[
            jax.ShapeDtypeStruct((n, c), jnp.float32),
            jax.ShapeDtypeStruct((n, 16), jnp.float32),
        ],
    )(x2d, W_gat, att_mat)


def _mm_c_body(v_ref, w_ref, b_ref, y_ref):
    v = v_ref[...]
    w = w_ref[...]
    y = lax.dot_general(v, w, (((0,), (1,)), ((), ())),
                        preferred_element_type=jnp.float32,
                        precision=lax.Precision.HIGHEST)
    y_ref[...] = y + b_ref[...]


def _matmul_c(V, fc_W, fc_b, bn=1024):
    c, n = V.shape
    return pl.pallas_call(
        _mm_c_body,
        grid=(n // bn,),
        in_specs=[
            pl.BlockSpec((c, bn), lambda i: (0, i)),
            pl.BlockSpec(fc_W.shape, lambda i: (0, 0)),
            pl.BlockSpec((1, c), lambda i: (0, 0)),
        ],
        out_specs=pl.BlockSpec((bn, c), lambda i: (i, 0)),
        out_shape=jax.ShapeDtypeStruct((n, c), jnp.float32),
    )(V, fc_W, fc_b.reshape(1, -1))


def kernel(x, edge_index, W_gat, att_src, att_dst, bias_gat, fc_W, fc_b):
    B, F_in, N, _ = x.shape
    K = edge_index.shape[-1]
    H = HEADS
    C_out = W_gat.shape[1]
    C = C_out // H
    NP = ((N + 1023) // 1024) * 1024

    x2d = x.reshape(F_in, N)
    x2d_p = jnp.pad(x2d, ((0, 0), (0, NP - N)))
    nn_p = jnp.pad(edge_index[0].reshape(N, K), ((0, NP - N), (0, 0)))

    eye = jnp.eye(H, dtype=jnp.float32)
    m_src = (eye[:, None, :] * att_src[:, :, None]).reshape(C_out, H)
    m_dst = (eye[:, None, :] * att_dst[:, :, None]).reshape(C_out, H)
    att_mat = jnp.concatenate([m_src, m_dst], axis=1)

    h_p, a_p = _matmul_a(x2d_p, W_gat, att_mat)
    asrcT = a_p[:, :H].T.reshape(-1)
    adstT = a_p[:, H:].T.reshape(-1)

    g = _gat_sc(h_p, asrcT, adstT, nn_p.reshape(-1), bias_gat)

    V = g.reshape(N * C_out).reshape(C_out, N)
    V_p = jnp.pad(V, ((0, 0), (0, NP - N)))
    y = _matmul_c(V_p, fc_W, fc_b)[:N]
    s = int(N ** 0.5)
    return y.reshape(B, C_out, s, s)
